# baseline (speedup 1.0000x reference)
import jax
import jax.numpy as jnp
from jax import lax
from jax.experimental import pallas as pl
from jax.experimental.pallas import tpu as pltpu
from jax.experimental.pallas import tpu_sc as plsc

NC, NS = 2, 16
NW = NC * NS
ROWS_PER_XFER = 128
NSLOT = 2
R_STREAM = 48
G_BLOCKS = R_STREAM * 200 // ROWS_PER_XFER
RSTAGE = 8
VOCAB = 5
L_SEQ = 200
EMB = 128
UNROLL = 16


def _tc_body(seqs_ref, cell_emb_ref, table_ref, w_ref, b_ref,
             cell_ref, comb_ref, idx_ref):
    bblk, L = seqs_ref.shape
    i = pl.program_id(0)

    cell = lax.dot_general(
        cell_emb_ref[...], w_ref[...],
        dimension_numbers=(((1,), (1,)), ((), ())),
        preferred_element_type=jnp.float32,
    ) + b_ref[...]
    cell_ref[...] = cell

    vocab = comb_ref.shape[1]
    comb_ref[...] = table_ref[:vocab][None, :, :] + cell[:, None, :]

    row = i * bblk + lax.broadcasted_iota(jnp.int32, (bblk, L), 0)
    idx_ref[...] = vocab * row + seqs_ref[...]


def _sc_body(comb_hbm, idx_hbm, seqs_hbm, out_hbm,
             idx_v, seq_v, comb_v, ebuf,
             gbuf0, gbuf1, gsem0, gsem1, ssem0, ssem1, esem):
    gbufs = (gbuf0, gbuf1)
    gsems = (gsem0, gsem1)
    ssems = (ssem0, ssem1)
    wid = lax.axis_index("s") * NC + lax.axis_index("c")
    rows_per_w = seqs_hbm.shape[0] // NW
    row_g0 = wid * rows_per_w
    base = row_g0 * L_SEQ

    pltpu.sync_copy(idx_hbm.at[wid, pl.ds(0, (G_BLOCKS + 7) // 8 * 8)], idx_v)

    def scatter_wait(buf, sem, n_rows):
        pltpu.make_async_copy(
            buf, out_hbm.at[pl.ds(base, n_rows)], sem).wait()

    def expand_row(seq_row, comb_base, r):
        def tok(l, src):
            for k in range(EMB // 16):
                ebuf[l, pl.ds(16 * k, 16)] = comb_v[src, pl.ds(16 * k, 16)]

        def grp(g, carry):
            del carry
            tok0 = g * UNROLL
            sv = seq_v[seq_row, pl.ds(tok0, UNROLL)]
            for u in range(UNROLL):
                tok(tok0 + u, comb_base + sv[u])
            return 0
        lax.fori_loop(0, L_SEQ // UNROLL, grp, 0)
        rem = L_SEQ % UNROLL
        if rem:
            sv = seq_v[seq_row, pl.ds(L_SEQ - UNROLL, UNROLL)]
            for u in range(UNROLL - rem, UNROLL):
                tok(L_SEQ - UNROLL + u, comb_base + sv[u])
        pltpu.async_copy(
            ebuf, out_hbm.at[pl.ds(base + r * L_SEQ, L_SEQ)], esem)

    n_iter = (rows_per_w - R_STREAM) // 2

    def body(it, carry):
        del carry
        for s in range(NSLOT):
            j = NSLOT * it + s

            @pl.when(j < G_BLOCKS)
            def _(s=s, j=j):
                @pl.when(j >= NSLOT)
                def _():
                    scatter_wait(gbufs[s], ssems[s], ROWS_PER_XFER)

                pltpu.async_copy(comb_hbm.at[idx_v.at[j]], gbufs[s],
                                 gsems[s])

            t = j - 1
            q = (s - 1) % NSLOT

            @pl.when(jnp.logical_and(t >= 0, t < G_BLOCKS))
            def _(q=q, t=t):
                pltpu.make_async_copy(
                    comb_hbm.at[idx_v.at[0]], gbufs[q], gsems[q]).wait()
                pltpu.async_copy(
                    gbufs[q],
                    out_hbm.at[pl.ds(base + t * ROWS_PER_XFER,
                                     ROWS_PER_XFER)],
                    ssems[q])

        @pl.when(it % (RSTAGE // 2) == 0)
        def _():
            rbase = row_g0 + R_STREAM + (it // (RSTAGE // 2)) * RSTAGE
            pltpu.sync_copy(seqs_hbm.at[pl.ds(rbase, RSTAGE)], seq_v)
            pltpu.sync_copy(
                comb_hbm.at[pl.ds(rbase * VOCAB, RSTAGE * VOCAB)], comb_v)

        for rr in range(2):
            er = 2 * it + rr
            r = R_STREAM + er
            seq_row = er % RSTAGE

            @pl.when(er > 0)
            def _():
                scatter_wait(ebuf, esem, L_SEQ)

            expand_row(seq_row, seq_row * VOCAB, r)

        return 0

    lax.fori_loop(0, n_iter, body, 0)

    for s in range(NSLOT):
        scatter_wait(gbufs[s], ssems[s], ROWS_PER_XFER)
    scatter_wait(ebuf, esem, L_SEQ)


def kernel(seqs, cell_emb, table, W, b):
    B, L = seqs.shape
    vocab, emb = table.shape
    cin = cell_emb.shape[1]

    vpad = 8
    table_p = jnp.zeros((vpad, emb), jnp.float32).at[:vocab].set(table)
    b2 = b.reshape(1, emb)

    BBLK = 512
    cell, comb, idx = pl.pallas_call(
        _tc_body,
        grid=(B // BBLK,),
        in_specs=[
            pl.BlockSpec((BBLK, L), lambda i: (i, 0)),
            pl.BlockSpec((BBLK, cin), lambda i: (i, 0)),
            pl.BlockSpec((vpad, emb), lambda i: (0, 0)),
            pl.BlockSpec((emb, cin), lambda i: (0, 0)),
            pl.BlockSpec((1, emb), lambda i: (0, 0)),
        ],
        out_specs=[
            pl.BlockSpec((BBLK, emb), lambda i: (i, 0)),
            pl.BlockSpec((BBLK, vocab, emb), lambda i: (i, 0, 0)),
            pl.BlockSpec((BBLK, L), lambda i: (i, 0)),
        ],
        out_shape=[
            jax.ShapeDtypeStruct((B, emb), jnp.float32),
            jax.ShapeDtypeStruct((B, vocab, emb), jnp.float32),
            jax.ShapeDtypeStruct((B, L), jnp.int32),
        ],
    )(seqs, cell_emb, table_p, W, b2)

    comb_flat = comb.reshape(B * vocab, emb)
    tokens = B * L
    n_xfer = tokens // (NW * ROWS_PER_XFER)
    idx3 = idx.reshape(NW, n_xfer, ROWS_PER_XFER)

    mesh = plsc.VectorSubcoreMesh(core_axis_name="c", subcore_axis_name="s")
    total_flat = pl.kernel(
        _sc_body,
        out_type=jax.ShapeDtypeStruct((tokens, emb), jnp.float32),
        mesh=mesh,
        scratch_types=(
            [pltpu.VMEM(((G_BLOCKS + 7) // 8 * 8, ROWS_PER_XFER), jnp.int32),
             pltpu.VMEM((RSTAGE, L), jnp.int32),
             pltpu.VMEM((RSTAGE * VOCAB, emb), jnp.float32),
             pltpu.VMEM((L, emb), jnp.float32)]
            + [pltpu.VMEM((ROWS_PER_XFER, emb), jnp.float32)] * NSLOT
            + [pltpu.SemaphoreType.DMA] * (2 * NSLOT + 1)
        ),
    )(comb_flat, idx3, seqs)

    return (total_flat.reshape(B, L, emb), cell)


# speedup vs baseline: 1.0260x; 1.0260x over previous
import jax
import jax.numpy as jnp
from jax import lax
from jax.experimental import pallas as pl
from jax.experimental.pallas import tpu as pltpu
from jax.experimental.pallas import tpu_sc as plsc

NC, NS = 2, 16
NW = NC * NS
ROWS_PER_XFER = 128
NSLOT = 3
RING_STEPS = 6
R_STREAM = 80
G_BLOCKS = R_STREAM * 200 // ROWS_PER_XFER
RSTAGE = 8
VOCAB = 5
L_SEQ = 200
EMB = 128
UNROLL = 16


def _tc_body(seqs_ref, cell_emb_ref, table_ref, w_ref, b_ref,
             cell_ref, comb_ref, idx_ref):
    bblk, L = seqs_ref.shape
    i = pl.program_id(0)

    cell = lax.dot_general(
        cell_emb_ref[...], w_ref[...],
        dimension_numbers=(((1,), (1,)), ((), ())),
        preferred_element_type=jnp.float32,
    ) + b_ref[...]
    cell_ref[...] = cell

    vocab = comb_ref.shape[1]
    comb_ref[...] = table_ref[:vocab][None, :, :] + cell[:, None, :]

    row = i * bblk + lax.broadcasted_iota(jnp.int32, (bblk, L), 0)
    idx_ref[...] = vocab * row + seqs_ref[...]


def _sc_body(comb_hbm, idx_hbm, seqs_hbm, out_hbm,
             idx_v, seq_v, comb_v, ebuf,
             gbuf0, gbuf1, gbuf2, gsem0, gsem1, gsem2,
             ssem0, ssem1, ssem2, esem):
    gbufs = (gbuf0, gbuf1, gbuf2)
    gsems = (gsem0, gsem1, gsem2)
    ssems = (ssem0, ssem1, ssem2)
    wid = lax.axis_index("s") * NC + lax.axis_index("c")
    rows_per_w = seqs_hbm.shape[0] // NW
    row_g0 = wid * rows_per_w
    base = row_g0 * L_SEQ

    pltpu.sync_copy(idx_hbm.at[wid, pl.ds(0, (G_BLOCKS + 7) // 8 * 8)], idx_v)

    def scatter_wait(buf, sem, n_rows):
        pltpu.make_async_copy(
            buf, out_hbm.at[pl.ds(base, n_rows)], sem).wait()

    def expand_row(seq_row, comb_base, r):
        def tok(l, src):
            for k in range(EMB // 16):
                ebuf[l, pl.ds(16 * k, 16)] = comb_v[src, pl.ds(16 * k, 16)]

        def grp(g, carry):
            del carry
            tok0 = g * UNROLL
            sv = seq_v[seq_row, pl.ds(tok0, UNROLL)]
            for u in range(UNROLL):
                tok(tok0 + u, comb_base + sv[u])
            return 0
        lax.fori_loop(0, L_SEQ // UNROLL, grp, 0)
        rem = L_SEQ % UNROLL
        if rem:
            sv = seq_v[seq_row, pl.ds(L_SEQ - UNROLL, UNROLL)]
            for u in range(UNROLL - rem, UNROLL):
                tok(L_SEQ - UNROLL + u, comb_base + sv[u])
        pltpu.async_copy(
            ebuf, out_hbm.at[pl.ds(base + r * L_SEQ, L_SEQ)], esem)

    n_iter = (rows_per_w - R_STREAM) // 2

    def body(it, carry):
        del carry
        for step in range(RING_STEPS):
            s = step % NSLOT
            j = RING_STEPS * it + step

            @pl.when(j < G_BLOCKS)
            def _(s=s, j=j):
                @pl.when(j >= NSLOT)
                def _():
                    scatter_wait(gbufs[s], ssems[s], ROWS_PER_XFER)

                pltpu.async_copy(comb_hbm.at[idx_v.at[j]], gbufs[s],
                                 gsems[s])

            t = j - 1
            q = (s - 1) % NSLOT

            @pl.when(jnp.logical_and(t >= 0, t < G_BLOCKS))
            def _(q=q, t=t):
                pltpu.make_async_copy(
                    comb_hbm.at[idx_v.at[0]], gbufs[q], gsems[q]).wait()
                pltpu.async_copy(
                    gbufs[q],
                    out_hbm.at[pl.ds(base + t * ROWS_PER_XFER,
                                     ROWS_PER_XFER)],
                    ssems[q])

        @pl.when(it % (RSTAGE // 2) == 0)
        def _():
            rbase = row_g0 + R_STREAM + (it // (RSTAGE // 2)) * RSTAGE
            pltpu.sync_copy(seqs_hbm.at[pl.ds(rbase, RSTAGE)], seq_v)
            pltpu.sync_copy(
                comb_hbm.at[pl.ds(rbase * VOCAB, RSTAGE * VOCAB)], comb_v)

        for rr in range(2):
            er = 2 * it + rr
            r = R_STREAM + er
            seq_row = er % RSTAGE

            @pl.when(er > 0)
            def _():
                scatter_wait(ebuf, esem, L_SEQ)

            expand_row(seq_row, seq_row * VOCAB, r)

        return 0

    lax.fori_loop(0, n_iter, body, 0)

    for s in range(NSLOT):
        scatter_wait(gbufs[s], ssems[s], ROWS_PER_XFER)
    scatter_wait(ebuf, esem, L_SEQ)


def kernel(seqs, cell_emb, table, W, b):
    B, L = seqs.shape
    vocab, emb = table.shape
    cin = cell_emb.shape[1]

    vpad = 8
    table_p = jnp.zeros((vpad, emb), jnp.float32).at[:vocab].set(table)
    b2 = b.reshape(1, emb)

    BBLK = 512
    cell, comb, idx = pl.pallas_call(
        _tc_body,
        grid=(B // BBLK,),
        in_specs=[
            pl.BlockSpec((BBLK, L), lambda i: (i, 0)),
            pl.BlockSpec((BBLK, cin), lambda i: (i, 0)),
            pl.BlockSpec((vpad, emb), lambda i: (0, 0)),
            pl.BlockSpec((emb, cin), lambda i: (0, 0)),
            pl.BlockSpec((1, emb), lambda i: (0, 0)),
        ],
        out_specs=[
            pl.BlockSpec((BBLK, emb), lambda i: (i, 0)),
            pl.BlockSpec((BBLK, vocab, emb), lambda i: (i, 0, 0)),
            pl.BlockSpec((BBLK, L), lambda i: (i, 0)),
        ],
        out_shape=[
            jax.ShapeDtypeStruct((B, emb), jnp.float32),
            jax.ShapeDtypeStruct((B, vocab, emb), jnp.float32),
            jax.ShapeDtypeStruct((B, L), jnp.int32),
        ],
    )(seqs, cell_emb, table_p, W, b2)

    comb_flat = comb.reshape(B * vocab, emb)
    tokens = B * L
    n_xfer = tokens // (NW * ROWS_PER_XFER)
    idx3 = idx.reshape(NW, n_xfer, ROWS_PER_XFER)

    mesh = plsc.VectorSubcoreMesh(core_axis_name="c", subcore_axis_name="s")
    total_flat = pl.kernel(
        _sc_body,
        out_type=jax.ShapeDtypeStruct((tokens, emb), jnp.float32),
        mesh=mesh,
        scratch_types=(
            [pltpu.VMEM(((G_BLOCKS + 7) // 8 * 8, ROWS_PER_XFER), jnp.int32),
             pltpu.VMEM((RSTAGE, L), jnp.int32),
             pltpu.VMEM((RSTAGE * VOCAB, emb), jnp.float32),
             pltpu.VMEM((L, emb), jnp.float32)]
            + [pltpu.VMEM((ROWS_PER_XFER, emb), jnp.float32)] * NSLOT
            + [pltpu.SemaphoreType.DMA] * (2 * NSLOT + 1)
        ),
    )(comb_flat, idx3, seqs)

    return (total_flat.reshape(B, L, emb), cell)


# speedup vs baseline: 1.2994x; 1.2664x over previous
import jax
import jax.numpy as jnp
from jax import lax
from jax.experimental import pallas as pl
from jax.experimental.pallas import tpu as pltpu
from jax.experimental.pallas import tpu_sc as plsc

NC, NS = 2, 16
NW = NC * NS
ROWS_PER_XFER = 128
NSLOT = 6
LAG = 5


def _tc_body(seqs_ref, cell_emb_ref, table_ref, w_ref, b_ref,
             cell_ref, comb_ref, idx_ref):
    bblk, L = seqs_ref.shape
    i = pl.program_id(0)

    cell = lax.dot_general(
        cell_emb_ref[...], w_ref[...],
        dimension_numbers=(((1,), (1,)), ((), ())),
        preferred_element_type=jnp.float32,
    ) + b_ref[...]
    cell_ref[...] = cell

    vocab = comb_ref.shape[1]
    comb_ref[...] = table_ref[:vocab][None, :, :] + cell[:, None, :]

    row = i * bblk + lax.broadcasted_iota(jnp.int32, (bblk, L), 0)
    idx_ref[...] = vocab * row + seqs_ref[...]


def _sc_body(comb_hbm, idx_hbm, out_hbm, idx_v, *bufsems):
    bufs = bufsems[:NSLOT]
    gsems = bufsems[NSLOT:2 * NSLOT]
    ssems = bufsems[2 * NSLOT:]
    wid = lax.axis_index("s") * NC + lax.axis_index("c")
    n_xfer = idx_hbm.shape[1]
    base = wid * n_xfer * ROWS_PER_XFER

    pltpu.sync_copy(idx_hbm.at[wid], idx_v)

    def scatter_wait(p, sem):
        pltpu.make_async_copy(
            bufs[p], out_hbm.at[pl.ds(base, ROWS_PER_XFER)], sem).wait()

    def round_(jj, carry):
        del carry
        j0 = jj * NSLOT
        for p in range(NSLOT):
            j = j0 + p
            @pl.when(j < n_xfer)
            def _(p=p, j=j):
                @pl.when(j >= NSLOT)
                def _():
                    scatter_wait(p, ssems[p])

                pltpu.async_copy(comb_hbm.at[idx_v.at[j]], bufs[p], gsems[p])

            t = j - LAG
            q = (p - LAG) % NSLOT

            @pl.when(jnp.logical_and(t >= 0, t < n_xfer))
            def _(q=q, t=t):
                pltpu.make_async_copy(
                    comb_hbm.at[idx_v.at[0]], bufs[q], gsems[q]).wait()
                pltpu.async_copy(
                    bufs[q],
                    out_hbm.at[pl.ds(base + t * ROWS_PER_XFER,
                                     ROWS_PER_XFER)],
                    ssems[q])

        return 0

    n_rounds = (n_xfer + LAG + NSLOT - 1) // NSLOT
    lax.fori_loop(0, n_rounds, round_, 0)

    for p in range(NSLOT):
        scatter_wait(p, ssems[p])


def kernel(seqs, cell_emb, table, W, b):
    B, L = seqs.shape
    vocab, emb = table.shape
    cin = cell_emb.shape[1]

    vpad = 8
    table_p = jnp.zeros((vpad, emb), jnp.float32).at[:vocab].set(table)
    b2 = b.reshape(1, emb)

    BBLK = 512
    cell, comb, idx = pl.pallas_call(
        _tc_body,
        grid=(B // BBLK,),
        in_specs=[
            pl.BlockSpec((BBLK, L), lambda i: (i, 0)),
            pl.BlockSpec((BBLK, cin), lambda i: (i, 0)),
            pl.BlockSpec((vpad, emb), lambda i: (0, 0)),
            pl.BlockSpec((emb, cin), lambda i: (0, 0)),
            pl.BlockSpec((1, emb), lambda i: (0, 0)),
        ],
        out_specs=[
            pl.BlockSpec((BBLK, emb), lambda i: (i, 0)),
            pl.BlockSpec((BBLK, vocab, emb), lambda i: (i, 0, 0)),
            pl.BlockSpec((BBLK, L), lambda i: (i, 0)),
        ],
        out_shape=[
            jax.ShapeDtypeStruct((B, emb), jnp.float32),
            jax.ShapeDtypeStruct((B, vocab, emb), jnp.float32),
            jax.ShapeDtypeStruct((B, L), jnp.int32),
        ],
    )(seqs, cell_emb, table_p, W, b2)

    comb_flat = comb.reshape(B * vocab, emb)
    tokens = B * L
    n_xfer = tokens // (NW * ROWS_PER_XFER)
    idx3 = idx.reshape(NW, n_xfer, ROWS_PER_XFER)

    mesh = plsc.VectorSubcoreMesh(core_axis_name="c", subcore_axis_name="s")
    total_flat = pl.kernel(
        _sc_body,
        out_type=jax.ShapeDtypeStruct((tokens, emb), jnp.float32),
        mesh=mesh,
        scratch_types=(
            [pltpu.VMEM((n_xfer, ROWS_PER_XFER), jnp.int32)]
            + [pltpu.VMEM((ROWS_PER_XFER, emb), jnp.float32)] * NSLOT
            + [pltpu.SemaphoreType.DMA] * (2 * NSLOT)
        ),
    )(comb_flat, idx3)

    return (total_flat.reshape(B, L, emb), cell)
